# baseline (speedup 1.0000x reference)
import jax
import jax.numpy as jnp
from jax import lax
from jax.experimental import pallas as pl
from jax.experimental.pallas import tpu as pltpu
from jax.experimental.pallas import tpu_sc as plsc
import functools

RANK = 32
NC = 2
NS = 16
NW = NC * NS
LANES = 16
GRP = 128
CHUNK = 512

_BITREV = [int(f"{l:04b}"[::-1], 2) for l in range(16)]

_GATHER_DNUMS = lax.GatherDimensionNumbers(
    offset_dims=(), collapsed_slice_dims=(0,), start_index_map=(0,))


def _lane_shuffle(x, idx):
    return lax.gather(
        x, idx[:, None], _GATHER_DNUMS, slice_sizes=(1,),
        mode=lax.GatherScatterMode.PROMISE_IN_BOUNDS)


TW = 800


def _transpose_kernel(n_rows):
    n_blocks = n_rows // TW
    n_t = (n_blocks + NW - 1) // NW

    mesh = plsc.VectorSubcoreMesh(core_axis_name="c", subcore_axis_name="s")

    @functools.partial(
        pl.kernel,
        out_type=(jax.ShapeDtypeStruct((n_rows, RANK), jnp.float32),
                  jax.ShapeDtypeStruct((n_rows, RANK), jnp.float32)),
        mesh=mesh,
        scratch_types=[
            pltpu.VMEM((2, RANK, TW), jnp.float32),
            pltpu.VMEM((2, TW, RANK), jnp.float32),
            pltpu.SemaphoreType.DMA,
            pltpu.SemaphoreType.DMA,
            pltpu.SemaphoreType.DMA,
            pltpu.SemaphoreType.DMA,
        ],
        compiler_params=pltpu.CompilerParams(
            use_tc_tiling_on_sc=False, needs_layout_passes=False),
    )
    def kern(ut_hbm, vt_hbm, urm_hbm, vrm_hbm, ibuf, obuf, isem0, isem1,
             osem0, osem1):
        wid = lax.axis_index("s") * NC + lax.axis_index("c")
        isems = (isem0, isem1)
        osems = (osem0, osem1)
        srcs = (ut_hbm, vt_hbm)
        dsts = (urm_hbm, vrm_hbm)
        iota = lax.broadcasted_iota(jnp.int32, (LANES,), 0)

        def in_copies(t, tab, slot):
            c0 = (wid + NW * t) * TW
            for k in range(RANK):
                yield pltpu.make_async_copy(
                    srcs[tab].at[k, pl.ds(c0, TW)], ibuf.at[slot, k],
                    isems[slot])

        def out_copy(t, tab, slot):
            c0 = (wid + NW * t) * TW
            return pltpu.make_async_copy(
                obuf.at[slot], dsts[tab].at[pl.ds(c0, TW)], osems[slot])

        def unit_valid(t):
            return wid + NW * t < n_blocks

        def issue_in(t, tab, slot):
            @pl.when(unit_valid(t))
            def _():
                for c in in_copies(t, tab, slot):
                    c.start()

        def transpose_block(slot):
            def col_body(g, _):
                c1 = g * LANES
                for h in range(RANK // LANES):
                    kidx = iota + (h * LANES)
                    for j in range(LANES):
                        tj = (iota + j) & (LANES - 1)
                        cidx = tj + c1
                        d = plsc.load_gather(ibuf.at[slot], [kidx, cidx])
                        plsc.store_scatter(obuf.at[slot], [cidx, kidx], d)
                return ()

            lax.fori_loop(0, TW // LANES, col_body, (), unroll=False)

        issue_in(0, 0, 0)

        def t_body(t, _):
            for tab in range(2):
                slot = tab
                nt_, ntab = (t, 1) if tab == 0 else (t + 1, 0)
                issue_in(nt_, ntab, 1 - slot)

                @pl.when(unit_valid(t))
                def _():
                    for c in in_copies(t, tab, slot):
                        c.wait()

                    @pl.when(t >= 1)
                    def _():
                        out_copy(t - 1, tab, slot).wait()

                    transpose_block(slot)
                    out_copy(t, tab, slot).start()
            return ()

        lax.fori_loop(0, n_t, t_body, (), unroll=False)
        t_last = (n_blocks - wid + NW - 1) // NW - 1
        for tab in range(2):
            out_copy(t_last, tab, tab).wait()

    return kern


def _factor_kernel(n_total):
    n_per_w = n_total // NW
    n_chunks = n_per_w // CHUNK
    jrows = CHUNK // GRP
    idx_rows = n_per_w // GRP

    mesh = plsc.VectorSubcoreMesh(core_axis_name="c", subcore_axis_name="s")

    @functools.partial(
        pl.kernel,
        out_type=jax.ShapeDtypeStruct((n_total,), jnp.float32),
        mesh=mesh,
        scratch_types=[
            pltpu.VMEM((idx_rows, GRP), jnp.int32),
            pltpu.VMEM((idx_rows, GRP), jnp.int32),
            pltpu.VMEM((2, CHUNK, RANK), jnp.float32),
            pltpu.VMEM((2, CHUNK, RANK), jnp.float32),
            pltpu.VMEM((2, CHUNK), jnp.float32),
            pltpu.SemaphoreType.DMA,
            pltpu.SemaphoreType.DMA,
            pltpu.SemaphoreType.DMA,
            pltpu.SemaphoreType.DMA,
        ],
        compiler_params=pltpu.CompilerParams(use_tc_tiling_on_sc=False),
    )
    def kern(rid_hbm, cid_hbm, u_hbm, v_hbm, out_hbm, ridx, cidx, urows,
             vrows, obuf, gsem0, gsem1, osem0, osem1):
        wid = lax.axis_index("s") * NC + lax.axis_index("c")
        base_row = wid * idx_rows
        out_base = wid * n_per_w
        gsems = (gsem0, gsem1)
        osems = (osem0, osem1)
        iota = lax.broadcasted_iota(jnp.int32, (LANES,), 0)

        pltpu.sync_copy(rid_hbm.at[pl.ds(base_row, idx_rows)], ridx)
        pltpu.sync_copy(cid_hbm.at[pl.ds(base_row, idx_rows)], cidx)

        def gather_chunk(g, slot):
            for j in range(jrows):
                pltpu.async_copy(
                    u_hbm.at[ridx.at[g * jrows + j]],
                    urows.at[slot, pl.ds(j * GRP, GRP)], gsems[slot])
                pltpu.async_copy(
                    v_hbm.at[cidx.at[g * jrows + j]],
                    vrows.at[slot, pl.ds(j * GRP, GRP)], gsems[slot])

        def wait_chunk(g, slot):
            for j in range(jrows):
                pltpu.make_async_copy(
                    u_hbm.at[ridx.at[g * jrows + j]],
                    urows.at[slot, pl.ds(j * GRP, GRP)], gsems[slot]).wait()
                pltpu.make_async_copy(
                    v_hbm.at[cidx.at[g * jrows + j]],
                    vrows.at[slot, pl.ds(j * GRP, GRP)], gsems[slot]).wait()

        def out_copy(g, slot):
            return pltpu.make_async_copy(
                obuf.at[slot],
                out_hbm.at[pl.ds(out_base + g * CHUNK, CHUNK)], osems[slot])

        def compute_chunk(g, slot):
            def dot_body(i, _):
                ss = []
                for c in range(LANES):
                    r = i * LANES + _BITREV[c]
                    u0 = urows[slot, r, pl.ds(0, LANES)]
                    u1 = urows[slot, r, pl.ds(LANES, LANES)]
                    v0 = vrows[slot, r, pl.ds(0, LANES)]
                    v1 = vrows[slot, r, pl.ds(LANES, LANES)]
                    ss.append(u0 * v0 + u1 * v1)
                for rbit in (8, 4, 2, 1):
                    rot = iota ^ rbit
                    keep = (iota & rbit) == 0
                    nxt = []
                    for k in range(0, len(ss), 2):
                        x, y = ss[k], ss[k + 1]
                        xr = _lane_shuffle(x, rot)
                        yr = _lane_shuffle(y, rot)
                        nxt.append(jnp.where(keep, x + xr, y + yr))
                    ss = nxt
                obuf[slot, pl.ds(i * LANES, LANES)] = ss[0]
                return ()

            lax.fori_loop(0, CHUNK // LANES, dot_body, (), unroll=False)

        gather_chunk(0, 0)

        def pair_body(i, _):
            for b in range(2):
                g = 2 * i + b
                slot = b

                @pl.when(g + 1 < n_chunks)
                def _():
                    gather_chunk(g + 1, 1 - slot)

                wait_chunk(g, slot)

                @pl.when(g >= 2)
                def _():
                    out_copy(g - 2, slot).wait()

                compute_chunk(g, slot)
                out_copy(g, slot).start()
            return ()

        lax.fori_loop(0, n_chunks // 2, pair_body, (), unroll=False)
        out_copy(n_chunks - 2, 0).wait()
        out_copy(n_chunks - 1, 1).wait()

    return kern


def kernel(row_ids, col_ids, U, V):
    b, l = row_ids.shape
    n = b * l
    rid = row_ids.reshape(n // GRP, GRP)
    cid = col_ids.reshape(n // GRP, GRP)
    urm, vrm = _transpose_kernel(U.shape[0])(U.T, V.T)
    out = _factor_kernel(n)(rid, cid, urm, vrm)
    return out.reshape(b, l)


# speedup vs baseline: 5.8906x; 5.8906x over previous
import jax
import jax.numpy as jnp
from jax import lax
from jax.experimental import pallas as pl
from jax.experimental.pallas import tpu as pltpu
from jax.experimental.pallas import tpu_sc as plsc
import functools

RANK = 32
NC = 2
NS = 16
NW = NC * NS
LANES = 16
GRP = 128
CHUNK = 128
TCOL = 128

_BITREV = [int(f"{l:04b}"[::-1], 2) for l in range(16)]

_GATHER_DNUMS = lax.GatherDimensionNumbers(
    offset_dims=(), collapsed_slice_dims=(0,), start_index_map=(0,))


def _lane_shuffle(x, idx):
    return lax.gather(
        x, idx[:, None], _GATHER_DNUMS, slice_sizes=(1,),
        mode=lax.GatherScatterMode.PROMISE_IN_BOUNDS)


def _transpose_kernel(n_rows):
    n_full = n_rows // TCOL
    n_rem = n_rows - n_full * TCOL
    n_t = (n_full + NW - 1) // NW
    out_rows = n_rows * RANK // 128

    mesh = plsc.VectorSubcoreMesh(core_axis_name="c", subcore_axis_name="s")

    @functools.partial(
        pl.kernel,
        out_type=(jax.ShapeDtypeStruct((out_rows, 128), jnp.float32),
                  jax.ShapeDtypeStruct((out_rows, 128), jnp.float32)),
        mesh=mesh,
        scratch_types=[
            pltpu.VMEM((2, RANK, TCOL), jnp.float32),
            pltpu.VMEM((2, TCOL * RANK // 128, 128), jnp.float32),
            pltpu.SemaphoreType.DMA,
            pltpu.SemaphoreType.DMA,
            pltpu.SemaphoreType.DMA,
            pltpu.SemaphoreType.DMA,
        ],
        compiler_params=pltpu.CompilerParams(needs_layout_passes=False),
    )
    def kern(ut_hbm, vt_hbm, urm_hbm, vrm_hbm, ibuf, obuf, isem0, isem1,
             osem0, osem1):
        wid = lax.axis_index("s") * NC + lax.axis_index("c")
        isems = (isem0, isem1)
        osems = (osem0, osem1)
        srcs = (ut_hbm, vt_hbm)
        dsts = (urm_hbm, vrm_hbm)
        iota = lax.broadcasted_iota(jnp.int32, (LANES,), 0)
        orows = TCOL * RANK // 128

        def in_copies(t, tab, slot):
            c0 = (wid + NW * t) * TCOL
            for h in range(RANK // 8):
                yield pltpu.make_async_copy(
                    srcs[tab].at[pl.ds(8 * h, 8), pl.ds(c0, TCOL)],
                    ibuf.at[slot, pl.ds(8 * h, 8)], isems[slot])

        def out_copy(t, tab, slot):
            r0 = (wid + NW * t) * orows
            return pltpu.make_async_copy(
                obuf.at[slot], dsts[tab].at[pl.ds(r0, orows)], osems[slot])

        def unit_valid(t):
            return wid + NW * t < n_full

        def issue_in(t, tab, slot):
            @pl.when(unit_valid(t))
            def _():
                for c in in_copies(t, tab, slot):
                    c.start()

        def transpose_block(slot, n_cols):
            def g_body(g, _):
                for h in range(RANK // LANES):
                    kidx = iota + (h * LANES)
                    for j in range(LANES):
                        tj = (iota + j) & (LANES - 1)
                        cidx = tj + (g * LANES)
                        d = plsc.load_gather(ibuf.at[slot], [kidx, cidx])
                        flat = ((cidx << 5) + kidx)
                        plsc.store_scatter(
                            obuf.at[slot],
                            [lax.shift_right_logical(flat, 7), flat & 127],
                            d)
                return ()

            lax.fori_loop(0, n_cols // LANES, g_body, (), unroll=False)

        issue_in(0, 0, 0)

        def t_body(t, _):
            for tab in range(2):
                slot = tab
                nt_, ntab = (t, 1) if tab == 0 else (t + 1, 0)
                issue_in(nt_, ntab, 1 - slot)

                @pl.when(unit_valid(t))
                def _():
                    for c in in_copies(t, tab, slot):
                        c.wait()

                    @pl.when(t >= 1)
                    def _():
                        out_copy(t - 1, tab, slot).wait()

                    transpose_block(slot, TCOL)
                    out_copy(t, tab, slot).start()
            return ()

        lax.fori_loop(0, n_t, t_body, (), unroll=False)
        t_last = (n_full - wid + NW - 1) // NW - 1
        for tab in range(2):
            out_copy(t_last, tab, tab).wait()

        if n_rem:
            @pl.when(wid == 0)
            def _():
                for tab in range(2):
                    for k in range(RANK):
                        pltpu.sync_copy(
                            srcs[tab].at[k, pl.ds(n_full * TCOL, n_rem)],
                            ibuf.at[0, k, pl.ds(0, n_rem)])
                    transpose_block(0, n_rem)
                    rr = n_rem * RANK // 128
                    pltpu.sync_copy(
                        obuf.at[0, pl.ds(0, rr)],
                        dsts[tab].at[pl.ds(n_full * orows, rr)])

    return kern


def _factor_kernel(n_total):
    n_per_w = n_total // NW
    n_chunks = n_per_w // CHUNK
    idx_rows = n_per_w // GRP

    mesh = plsc.VectorSubcoreMesh(core_axis_name="c", subcore_axis_name="s")

    @functools.partial(
        pl.kernel,
        out_type=jax.ShapeDtypeStruct((n_total,), jnp.float32),
        mesh=mesh,
        scratch_types=[
            pltpu.VMEM((idx_rows, GRP), jnp.int32),
            pltpu.VMEM((idx_rows, GRP), jnp.int32),
            pltpu.VMEM((2, 2, CHUNK), jnp.int32),
            pltpu.VMEM((2, CHUNK, 128), jnp.float32),
            pltpu.VMEM((2, CHUNK, 128), jnp.float32),
            pltpu.VMEM((2, CHUNK), jnp.float32),
            pltpu.SemaphoreType.DMA,
            pltpu.SemaphoreType.DMA,
            pltpu.SemaphoreType.DMA,
            pltpu.SemaphoreType.DMA,
        ],
        compiler_params=pltpu.CompilerParams(needs_layout_passes=False),
    )
    def kern(rid_hbm, cid_hbm, u_hbm, v_hbm, out_hbm, ridx, cidx, gidx,
             urows, vrows, obuf, gsem0, gsem1, osem0, osem1):
        wid = lax.axis_index("s") * NC + lax.axis_index("c")
        base_row = wid * idx_rows
        out_base = wid * n_per_w
        gsems = (gsem0, gsem1)
        osems = (osem0, osem1)
        iota = lax.broadcasted_iota(jnp.int32, (LANES,), 0)

        pltpu.sync_copy(rid_hbm.at[pl.ds(base_row, idx_rows)], ridx)
        pltpu.sync_copy(cid_hbm.at[pl.ds(base_row, idx_rows)], cidx)

        def fill_gidx(g, slot):
            for i in range(CHUNK // LANES):
                r = g * (CHUNK // GRP)
                ru = ridx[r, pl.ds(i * LANES, LANES)]
                cu = cidx[r, pl.ds(i * LANES, LANES)]
                gidx[slot, 0, pl.ds(i * LANES, LANES)] = (
                    lax.shift_right_logical(ru, 2))
                gidx[slot, 1, pl.ds(i * LANES, LANES)] = (
                    lax.shift_right_logical(cu, 2))

        def gather_chunk(g, slot):
            pltpu.async_copy(
                u_hbm.at[gidx.at[slot, 0]], urows.at[slot], gsems[slot])
            pltpu.async_copy(
                v_hbm.at[gidx.at[slot, 1]], vrows.at[slot], gsems[slot])

        def wait_chunk(g, slot):
            pltpu.make_async_copy(
                u_hbm.at[gidx.at[slot, 0]], urows.at[slot],
                gsems[slot]).wait()
            pltpu.make_async_copy(
                v_hbm.at[gidx.at[slot, 1]], vrows.at[slot],
                gsems[slot]).wait()

        def out_copy(g, slot):
            return pltpu.make_async_copy(
                obuf.at[slot],
                out_hbm.at[pl.ds(out_base + g * CHUNK, CHUNK)], osems[slot])

        def compute_chunk(g, slot):
            r = g * (CHUNK // GRP)
            for i in range(CHUNK // LANES):
                ru = ridx[r, pl.ds(i * LANES, LANES)]
                cu = cidx[r, pl.ds(i * LANES, LANES)]
                qu = (ru & 3) << 5
                qv = (cu & 3) << 5
                ss = []
                for c in range(LANES):
                    p = i * LANES + _BITREV[c]
                    oU = qu[_BITREV[c]]
                    oV = qv[_BITREV[c]]
                    u0 = urows[slot, p, pl.ds(oU, LANES)]
                    u1 = urows[slot, p, pl.ds(oU + LANES, LANES)]
                    v0 = vrows[slot, p, pl.ds(oV, LANES)]
                    v1 = vrows[slot, p, pl.ds(oV + LANES, LANES)]
                    ss.append(u0 * v0 + u1 * v1)
                for rbit in (8, 4, 2, 1):
                    rot = iota ^ rbit
                    keep = (iota & rbit) == 0
                    nxt = []
                    for k in range(0, len(ss), 2):
                        x, y = ss[k], ss[k + 1]
                        xr = _lane_shuffle(x, rot)
                        yr = _lane_shuffle(y, rot)
                        nxt.append(jnp.where(keep, x + xr, y + yr))
                    ss = nxt
                obuf[slot, pl.ds(i * LANES, LANES)] = ss[0]

        fill_gidx(0, 0)
        gather_chunk(0, 0)

        def pair_body(i, _):
            for b in range(2):
                g = 2 * i + b
                slot = b

                @pl.when(g + 1 < n_chunks)
                def _():
                    fill_gidx(g + 1, 1 - slot)
                    gather_chunk(g + 1, 1 - slot)

                wait_chunk(g, slot)

                @pl.when(g >= 2)
                def _():
                    out_copy(g - 2, slot).wait()

                compute_chunk(g, slot)
                out_copy(g, slot).start()
            return ()

        lax.fori_loop(0, n_chunks // 2, pair_body, (), unroll=False)
        out_copy(n_chunks - 2, 0).wait()
        out_copy(n_chunks - 1, 1).wait()

    return kern


def kernel(row_ids, col_ids, U, V):
    b, l = row_ids.shape
    n = b * l
    rid = row_ids.reshape(n // GRP, GRP)
    cid = col_ids.reshape(n // GRP, GRP)
    u4, v4 = _transpose_kernel(U.shape[0])(U.T, V.T)
    out = _factor_kernel(n)(rid, cid, u4, v4)
    return out.reshape(b, l)
